# baseline (speedup 1.0000x reference)
import functools

import jax
import jax.numpy as jnp
import numpy as np
from jax import lax
from jax.experimental import pallas as pl
from jax.experimental.pallas import tpu as pltpu
from jax.experimental.pallas import tpu_sc as plsc

EPS = 1e-5

NC = 2
NS = 16
NW = NC * NS
LANE = 16

SPLIT = 96
G = 2
IDXG = 16


def _pack_kernel(V, D):
    CHR = 120
    NCH = V // CHR
    mesh = plsc.VectorSubcoreMesh(
        core_axis_name="c", subcore_axis_name="s",
        num_cores=NC, num_subcores=NS)

    @functools.partial(
        pl.kernel,
        out_type=jax.ShapeDtypeStruct((V, D // 2), jnp.int32),
        mesh=mesh,
        scratch_types=[
            pltpu.VMEM((CHR, D), jnp.float32),
            pltpu.VMEM((CHR, D // 2), jnp.int32),
        ],
        compiler_params=pltpu.CompilerParams(
            use_tc_tiling_on_sc=False, needs_layout_passes=False),
    )
    def pack_tab(embed_hbm, out_hbm, fin_v, pk_v):
        wid = lax.axis_index("s") * NC + lax.axis_index("c")
        nt = (NCH + NW - 1) // NW

        def body(t, _):
            ch = t * NW + wid

            @pl.when(ch < NCH)
            def _():
                pltpu.sync_copy(embed_hbm.at[pl.ds(ch * CHR, CHR)], fin_v)

                def rb(r, _2):
                    for half in range(D // 32):
                        v0 = fin_v[r, pl.ds(half * 32, 16)]
                        v1 = fin_v[r, pl.ds(half * 32 + 16, 16)]
                        pk = plsc.pack(v0, v1,
                                       format=plsc.PackFormat.INTERLEAVED)
                        pk_v[r, pl.ds(half * 16, 16)] = plsc.bitcast(
                            pk, jnp.int32)
                    return 0

                lax.fori_loop(0, CHR, rb, 0)
                pltpu.sync_copy(pk_v, out_hbm.at[pl.ds(ch * CHR, CHR)])

            return 0

        lax.fori_loop(0, nt, body, 0)

    return pack_tab


def _pool_kernel(B, SEQ, V, D):
    SPW = B // NW
    NGRP = SPW // G
    NACC = D // LANE
    UNR = 8
    inv_seq = 1.0 / SEQ

    NBUF = 4
    mesh = plsc.VectorSubcoreMesh(
        core_axis_name="c", subcore_axis_name="s",
        num_cores=NC, num_subcores=NS)

    @functools.partial(
        pl.kernel,
        out_type=jax.ShapeDtypeStruct((B, D), jnp.float32),
        mesh=mesh,
        scratch_types=[
            pltpu.VMEM((2 * IDXG * G, SEQ), jnp.int32),
            pltpu.VMEM((NBUF, G * SEQ, D // 2), jnp.int32),
            pltpu.VMEM((SPW, D), jnp.float32),
            [pltpu.SemaphoreType.DMA] * NBUF,
        ],
        compiler_params=pltpu.CompilerParams(
            use_tc_tiling_on_sc=False, needs_layout_passes=False),
    )
    def pool(x_hbm, embed_hbm, out_hbm, idx_v, rows_v, pooled_v, sems):
        wid = lax.axis_index("s") * NC + lax.axis_index("c")
        wbase = wid * SPW

        def gather_copy(buf, g, s, h):
            irow = (g % (2 * IDXG)) * G + s
            off = h * SPLIT
            ln = SPLIT if h == 0 else SEQ - SPLIT
            return pltpu.make_async_copy(
                embed_hbm.at[idx_v.at[irow, pl.ds(off, ln)]],
                rows_v.at[buf, pl.ds(s * SEQ + off, ln)],
                sems[buf])

        def issue(buf, g):
            for s in range(G):
                for h in range(2):
                    gather_copy(buf, g, s, h).start()

        def drain(buf, g):
            for s in range(G):
                for h in range(2):
                    gather_copy(buf, g, s, h).wait()

        def refill(nb):
            blk = IDXG * G
            pltpu.sync_copy(
                x_hbm.at[pl.ds(wbase + nb * blk, blk)],
                idx_v.at[pl.ds((nb % 2) * blk, blk)])

        def accum_group(buf, g):
            rows = rows_v.at[buf]
            for s in range(G):
                base = s * SEQ

                def rbody(r, accs):
                    accs = list(accs)
                    for u in range(UNR // 2):
                        row = base + r * UNR + 2 * u
                        for v in range(D // 32):
                            sv = (plsc.bitcast(rows[row, pl.ds(v * 16, 16)],
                                               jnp.bfloat16)
                                  + plsc.bitcast(rows[row + 1, pl.ds(v * 16, 16)],
                                                 jnp.bfloat16))
                            even, odd = plsc.unpack(
                                sv, format=plsc.PackFormat.INTERLEAVED)
                            k = (u % 2) * NACC + 2 * v
                            accs[k] = accs[k] + even
                            accs[k + 1] = accs[k + 1] + odd
                    return tuple(accs)

                zero = jnp.zeros((LANE,), jnp.float32)
                accs = lax.fori_loop(0, SEQ // UNR, rbody, (zero,) * (2 * NACC))
                prow = g * G + s
                for c in range(NACC):
                    pooled_v[prow, pl.ds(c * LANE, LANE)] = (
                        (accs[c] + accs[NACC + c]) * inv_seq)

        refill(0)
        for k in range(NBUF - 1):
            issue(k, k)

        def outer(i, _):
            for k in range(NBUF):
                g = NBUF * i + k
                drain(k, g)
                gnext = g + (NBUF - 1)

                @pl.when(gnext < NGRP)
                def _():
                    @pl.when(gnext % IDXG == 0)
                    def _():
                        refill(gnext // IDXG)

                    issue((k + NBUF - 1) % NBUF, gnext)

                accum_group(k, g)
            return 0

        lax.fori_loop(0, NGRP // NBUF, outer, 0)
        pltpu.sync_copy(pooled_v, out_hbm.at[pl.ds(wid * SPW, SPW)])

    return pool


def _mlp_call(pooled, W1, b1r, gammar, betar, W2, b2r, B, D, H, CLS, blk):
    nblk = B // blk

    def body(p_ref, w1_ref, b1_ref, g_ref, be_ref, w2_ref, b2_ref,
             o_ref, sum_scr, s_scr, muh_scr, scale_scr):
        ph = pl.program_id(0)
        j = pl.program_id(1)

        @pl.when(ph == 0)
        def _():
            @pl.when(j == 0)
            def _():
                sum_scr[...] = jnp.zeros_like(sum_scr)
                s_scr[...] = jnp.zeros_like(s_scr)

            p = p_ref[...]
            sum_scr[...] += jnp.sum(p, axis=0, keepdims=True)
            s_scr[...] += lax.dot_general(
                p, p, (((0,), (0,)), ((), ())),
                preferred_element_type=jnp.float32)

            @pl.when(j == nblk - 1)
            def _():
                mu_p = sum_scr[...] * (1.0 / B)
                cov = s_scr[...] * (1.0 / B) - lax.dot_general(
                    mu_p, mu_p, (((0,), (0,)), ((), ())),
                    preferred_element_type=jnp.float32)
                w1 = w1_ref[...]
                muh_scr[...] = (
                    jnp.dot(mu_p, w1, preferred_element_type=jnp.float32)
                    + b1_ref[...])
                m = jnp.dot(cov, w1, preferred_element_type=jnp.float32)
                varh = jnp.sum(w1 * m, axis=0, keepdims=True)
                scale_scr[...] = g_ref[...] * lax.rsqrt(varh + EPS)

        @pl.when(ph == 1)
        def _():
            h = jnp.dot(p_ref[...], w1_ref[...],
                        preferred_element_type=jnp.float32) + b1_ref[...]
            hn = (h - muh_scr[...]) * scale_scr[...] + be_ref[...]
            hr = jnp.maximum(hn, 0.0)
            o_ref[...] = jnp.dot(
                hr, w2_ref[...], preferred_element_type=jnp.float32) + b2_ref[...]

    full = lambda ph, j: (0, 0)
    return pl.pallas_call(
        body,
        grid=(2, nblk),
        in_specs=[
            pl.BlockSpec((blk, D), lambda ph, j: (j, 0)),
            pl.BlockSpec((D, H), full),
            pl.BlockSpec((1, H), full),
            pl.BlockSpec((1, H), full),
            pl.BlockSpec((1, H), full),
            pl.BlockSpec((H, CLS), full),
            pl.BlockSpec((1, CLS), full),
        ],
        out_specs=pl.BlockSpec((blk, CLS), lambda ph, j: (j, 0)),
        out_shape=jax.ShapeDtypeStruct((B, CLS), jnp.float32),
        scratch_shapes=[
            pltpu.VMEM((1, D), jnp.float32),
            pltpu.VMEM((D, D), jnp.float32),
            pltpu.VMEM((1, H), jnp.float32),
            pltpu.VMEM((1, H), jnp.float32),
        ],
    )(pooled, W1, b1r, gammar, betar, W2, b2r)


def kernel(x, embed, W1, b1, gamma, beta, W2, b2):
    B, SEQ = x.shape
    V, D = embed.shape
    H = W1.shape[1]
    CLS = W2.shape[1]

    embed_pk = _pack_kernel(V, D)(embed)
    pooled = _pool_kernel(B, SEQ, V, D)(x.astype(jnp.int32), embed_pk)

    out = _mlp_call(
        pooled, W1, b1.reshape(1, H), gamma.reshape(1, H),
        beta.reshape(1, H), W2, b2.reshape(1, CLS), B, D, H, CLS, blk=1024)
    return out


# speedup vs baseline: 1.0308x; 1.0308x over previous
import functools

import jax
import jax.numpy as jnp
import numpy as np
from jax import lax
from jax.experimental import pallas as pl
from jax.experimental.pallas import tpu as pltpu
from jax.experimental.pallas import tpu_sc as plsc

EPS = 1e-5

NC = 2
NS = 16
NW = NC * NS
LANE = 16

SPLIT = 96
G = 2
IDXG = 16


def _pack_kernel(V, D):
    CHR = 120
    NCH = V // CHR
    mesh = plsc.VectorSubcoreMesh(
        core_axis_name="c", subcore_axis_name="s",
        num_cores=NC, num_subcores=NS)

    @functools.partial(
        pl.kernel,
        out_type=jax.ShapeDtypeStruct((V, D // 2), jnp.int32),
        mesh=mesh,
        scratch_types=[
            pltpu.VMEM((CHR, D), jnp.float32),
            pltpu.VMEM((CHR, D // 2), jnp.int32),
        ],
        compiler_params=pltpu.CompilerParams(
            use_tc_tiling_on_sc=False, needs_layout_passes=False),
    )
    def pack_tab(embed_hbm, out_hbm, fin_v, pk_v):
        wid = lax.axis_index("s") * NC + lax.axis_index("c")
        nt = (NCH + NW - 1) // NW

        def body(t, _):
            ch = t * NW + wid

            @pl.when(ch < NCH)
            def _():
                pltpu.sync_copy(embed_hbm.at[pl.ds(ch * CHR, CHR)], fin_v)

                def rb(r, _2):
                    for half in range(D // 32):
                        v0 = fin_v[r, pl.ds(half * 32, 16)]
                        v1 = fin_v[r, pl.ds(half * 32 + 16, 16)]
                        pk = plsc.pack(v0, v1,
                                       format=plsc.PackFormat.INTERLEAVED)
                        pk_v[r, pl.ds(half * 16, 16)] = plsc.bitcast(
                            pk, jnp.int32)
                    return 0

                lax.fori_loop(0, CHR, rb, 0)
                pltpu.sync_copy(pk_v, out_hbm.at[pl.ds(ch * CHR, CHR)])

            return 0

        lax.fori_loop(0, nt, body, 0)

    return pack_tab


def _pool_kernel(B, SEQ, V, D):
    SPW = B // NW
    NGRP = SPW // G
    NACC = D // LANE
    UNR = 4
    inv_seq = 1.0 / SEQ

    NBUF = 4
    mesh = plsc.VectorSubcoreMesh(
        core_axis_name="c", subcore_axis_name="s",
        num_cores=NC, num_subcores=NS)

    @functools.partial(
        pl.kernel,
        out_type=jax.ShapeDtypeStruct((B, D), jnp.float32),
        mesh=mesh,
        scratch_types=[
            pltpu.VMEM((2 * IDXG * G, SEQ), jnp.int32),
            pltpu.VMEM((NBUF, G * SEQ, D // 2), jnp.int32),
            pltpu.VMEM((SPW, D), jnp.float32),
            [pltpu.SemaphoreType.DMA] * NBUF,
        ],
        compiler_params=pltpu.CompilerParams(
            use_tc_tiling_on_sc=False, needs_layout_passes=False),
    )
    def pool(x_hbm, embed_hbm, out_hbm, idx_v, rows_v, pooled_v, sems):
        wid = lax.axis_index("s") * NC + lax.axis_index("c")
        wbase = wid * SPW

        def gather_copy(buf, g, s, h):
            irow = (g % (2 * IDXG)) * G + s
            off = h * SPLIT
            ln = SPLIT if h == 0 else SEQ - SPLIT
            return pltpu.make_async_copy(
                embed_hbm.at[idx_v.at[irow, pl.ds(off, ln)]],
                rows_v.at[buf, pl.ds(s * SEQ + off, ln)],
                sems[buf])

        def issue(buf, g):
            for s in range(G):
                for h in range(2):
                    gather_copy(buf, g, s, h).start()

        def drain(buf, g):
            for s in range(G):
                for h in range(2):
                    gather_copy(buf, g, s, h).wait()

        def refill(nb):
            blk = IDXG * G
            pltpu.sync_copy(
                x_hbm.at[pl.ds(wbase + nb * blk, blk)],
                idx_v.at[pl.ds((nb % 2) * blk, blk)])

        def accum_group(buf, g):
            rows = rows_v.at[buf]
            for s in range(G):
                base = s * SEQ

                def rbody(r, accs):
                    accs = list(accs)
                    for u in range(UNR // 2):
                        row = base + r * UNR + 2 * u
                        for v in range(D // 32):
                            sv = (plsc.bitcast(rows[row, pl.ds(v * 16, 16)],
                                               jnp.bfloat16)
                                  + plsc.bitcast(rows[row + 1, pl.ds(v * 16, 16)],
                                                 jnp.bfloat16))
                            even, odd = plsc.unpack(
                                sv, format=plsc.PackFormat.INTERLEAVED)
                            k = (u % 2) * NACC + 2 * v
                            accs[k] = accs[k] + even
                            accs[k + 1] = accs[k + 1] + odd
                    return tuple(accs)

                zero = jnp.zeros((LANE,), jnp.float32)
                accs = lax.fori_loop(0, SEQ // UNR, rbody, (zero,) * (2 * NACC))
                prow = g * G + s
                for c in range(NACC):
                    pooled_v[prow, pl.ds(c * LANE, LANE)] = (
                        (accs[c] + accs[NACC + c]) * inv_seq)

        refill(0)
        for k in range(NBUF - 1):
            issue(k, k)

        def outer(i, _):
            for k in range(NBUF):
                g = NBUF * i + k
                drain(k, g)
                gnext = g + (NBUF - 1)

                @pl.when(gnext < NGRP)
                def _():
                    @pl.when(gnext % IDXG == 0)
                    def _():
                        refill(gnext // IDXG)

                    issue((k + NBUF - 1) % NBUF, gnext)

                accum_group(k, g)
            return 0

        lax.fori_loop(0, NGRP // NBUF, outer, 0)
        pltpu.sync_copy(pooled_v, out_hbm.at[pl.ds(wid * SPW, SPW)])

    return pool


def _mlp_call(pooled, W1, b1r, gammar, betar, W2, b2r, B, D, H, CLS, blk):
    nblk = B // blk

    def body(p_ref, w1_ref, b1_ref, g_ref, be_ref, w2_ref, b2_ref,
             o_ref, sum_scr, s_scr, muh_scr, scale_scr):
        ph = pl.program_id(0)
        j = pl.program_id(1)

        @pl.when(ph == 0)
        def _():
            @pl.when(j == 0)
            def _():
                sum_scr[...] = jnp.zeros_like(sum_scr)
                s_scr[...] = jnp.zeros_like(s_scr)

            p = p_ref[...]
            sum_scr[...] += jnp.sum(p, axis=0, keepdims=True)
            s_scr[...] += lax.dot_general(
                p, p, (((0,), (0,)), ((), ())),
                preferred_element_type=jnp.float32)

            @pl.when(j == nblk - 1)
            def _():
                mu_p = sum_scr[...] * (1.0 / B)
                cov = s_scr[...] * (1.0 / B) - lax.dot_general(
                    mu_p, mu_p, (((0,), (0,)), ((), ())),
                    preferred_element_type=jnp.float32)
                w1 = w1_ref[...]
                muh_scr[...] = (
                    jnp.dot(mu_p, w1, preferred_element_type=jnp.float32)
                    + b1_ref[...])
                m = jnp.dot(cov, w1, preferred_element_type=jnp.float32)
                varh = jnp.sum(w1 * m, axis=0, keepdims=True)
                scale_scr[...] = g_ref[...] * lax.rsqrt(varh + EPS)

        @pl.when(ph == 1)
        def _():
            h = jnp.dot(p_ref[...], w1_ref[...],
                        preferred_element_type=jnp.float32) + b1_ref[...]
            hn = (h - muh_scr[...]) * scale_scr[...] + be_ref[...]
            hr = jnp.maximum(hn, 0.0)
            o_ref[...] = jnp.dot(
                hr, w2_ref[...], preferred_element_type=jnp.float32) + b2_ref[...]

    full = lambda ph, j: (0, 0)
    return pl.pallas_call(
        body,
        grid=(2, nblk),
        in_specs=[
            pl.BlockSpec((blk, D), lambda ph, j: (j, 0)),
            pl.BlockSpec((D, H), full),
            pl.BlockSpec((1, H), full),
            pl.BlockSpec((1, H), full),
            pl.BlockSpec((1, H), full),
            pl.BlockSpec((H, CLS), full),
            pl.BlockSpec((1, CLS), full),
        ],
        out_specs=pl.BlockSpec((blk, CLS), lambda ph, j: (j, 0)),
        out_shape=jax.ShapeDtypeStruct((B, CLS), jnp.float32),
        scratch_shapes=[
            pltpu.VMEM((1, D), jnp.float32),
            pltpu.VMEM((D, D), jnp.float32),
            pltpu.VMEM((1, H), jnp.float32),
            pltpu.VMEM((1, H), jnp.float32),
        ],
    )(pooled, W1, b1r, gammar, betar, W2, b2r)


def kernel(x, embed, W1, b1, gamma, beta, W2, b2):
    B, SEQ = x.shape
    V, D = embed.shape
    H = W1.shape[1]
    CLS = W2.shape[1]

    embed_pk = _pack_kernel(V, D)(embed)
    pooled = _pool_kernel(B, SEQ, V, D)(x.astype(jnp.int32), embed_pk)

    out = _mlp_call(
        pooled, W1, b1.reshape(1, H), gamma.reshape(1, H),
        beta.reshape(1, H), W2, b2.reshape(1, CLS), B, D, H, CLS, blk=2048)
    return out


# speedup vs baseline: 1.0508x; 1.0194x over previous
import functools

import jax
import jax.numpy as jnp
import numpy as np
from jax import lax
from jax.experimental import pallas as pl
from jax.experimental.pallas import tpu as pltpu
from jax.experimental.pallas import tpu_sc as plsc

EPS = 1e-5

NC = 2
NS = 16
NW = NC * NS
LANE = 16

SPLIT = 96
G = 2
IDXG = 16


def _pack_kernel(V, D):
    CHR = 120
    NCH = V // CHR
    mesh = plsc.VectorSubcoreMesh(
        core_axis_name="c", subcore_axis_name="s",
        num_cores=NC, num_subcores=NS)

    @functools.partial(
        pl.kernel,
        out_type=jax.ShapeDtypeStruct((V, D // 2), jnp.int32),
        mesh=mesh,
        scratch_types=[
            pltpu.VMEM((CHR, D), jnp.float32),
            pltpu.VMEM((CHR, D // 2), jnp.int32),
        ],
        compiler_params=pltpu.CompilerParams(
            use_tc_tiling_on_sc=False, needs_layout_passes=False),
    )
    def pack_tab(embed_hbm, out_hbm, fin_v, pk_v):
        wid = lax.axis_index("s") * NC + lax.axis_index("c")
        nt = (NCH + NW - 1) // NW

        def body(t, _):
            ch = t * NW + wid

            @pl.when(ch < NCH)
            def _():
                pltpu.sync_copy(embed_hbm.at[pl.ds(ch * CHR, CHR)], fin_v)

                def rb(r, _2):
                    for half in range(D // 32):
                        v0 = fin_v[r, pl.ds(half * 32, 16)]
                        v1 = fin_v[r, pl.ds(half * 32 + 16, 16)]
                        pk = plsc.pack(v0, v1,
                                       format=plsc.PackFormat.INTERLEAVED)
                        pk_v[r, pl.ds(half * 16, 16)] = plsc.bitcast(
                            pk, jnp.int32)
                    return 0

                lax.fori_loop(0, CHR, rb, 0)
                pltpu.sync_copy(pk_v, out_hbm.at[pl.ds(ch * CHR, CHR)])

            return 0

        lax.fori_loop(0, nt, body, 0)

    return pack_tab


def _pool_kernel(B, SEQ, V, D):
    SPW = B // NW
    NGRP = SPW // G
    NACC = D // LANE
    UNR = 4
    inv_seq = 1.0 / SEQ

    NBUF = 4
    mesh = plsc.VectorSubcoreMesh(
        core_axis_name="c", subcore_axis_name="s",
        num_cores=NC, num_subcores=NS)

    @functools.partial(
        pl.kernel,
        out_type=jax.ShapeDtypeStruct((B, D), jnp.float32),
        mesh=mesh,
        scratch_types=[
            pltpu.VMEM((2 * IDXG * G, SEQ), jnp.int32),
            pltpu.VMEM((NBUF, G * SEQ, D // 2), jnp.int32),
            pltpu.VMEM((SPW, D), jnp.float32),
            [pltpu.SemaphoreType.DMA] * NBUF,
        ],
        compiler_params=pltpu.CompilerParams(
            use_tc_tiling_on_sc=False, needs_layout_passes=False),
    )
    def pool(x_hbm, embed_hbm, out_hbm, idx_v, rows_v, pooled_v, sems):
        wid = lax.axis_index("s") * NC + lax.axis_index("c")
        wbase = wid * SPW

        def gather_copy(buf, g, s, h):
            irow = (g % (2 * IDXG)) * G + s
            off = h * SPLIT
            ln = SPLIT if h == 0 else SEQ - SPLIT
            return pltpu.make_async_copy(
                embed_hbm.at[idx_v.at[irow, pl.ds(off, ln)]],
                rows_v.at[buf, pl.ds(s * SEQ + off, ln)],
                sems[buf])

        def issue(buf, g):
            for s in range(G):
                for h in range(2):
                    gather_copy(buf, g, s, h).start()

        def drain(buf, g):
            for s in range(G):
                for h in range(2):
                    gather_copy(buf, g, s, h).wait()

        def refill(nb):
            blk = IDXG * G
            pltpu.sync_copy(
                x_hbm.at[pl.ds(wbase + nb * blk, blk)],
                idx_v.at[pl.ds((nb % 2) * blk, blk)])

        def accum_group(buf, g):
            rows = rows_v.at[buf]
            for s in range(G):
                base = s * SEQ

                def rbody(r, accs):
                    accs = list(accs)
                    for u in range(UNR // 2):
                        row = base + r * UNR + 2 * u
                        for v in range(D // 32):
                            sv = (plsc.bitcast(rows[row, pl.ds(v * 16, 16)],
                                               jnp.bfloat16)
                                  + plsc.bitcast(rows[row + 1, pl.ds(v * 16, 16)],
                                                 jnp.bfloat16))
                            even, odd = plsc.unpack(
                                sv, format=plsc.PackFormat.INTERLEAVED)
                            k = (u % 2) * NACC + 2 * v
                            accs[k] = accs[k] + even
                            accs[k + 1] = accs[k + 1] + odd
                    return tuple(accs)

                zero = jnp.zeros((LANE,), jnp.float32)
                accs = lax.fori_loop(0, SEQ // UNR, rbody, (zero,) * (2 * NACC))
                prow = g * G + s
                for c in range(NACC):
                    pooled_v[prow, pl.ds(c * LANE, LANE)] = (
                        (accs[c] + accs[NACC + c]) * inv_seq)

        refill(0)
        for k in range(NBUF - 1):
            issue(k, k)

        def outer(i, _):
            for k in range(NBUF):
                g = NBUF * i + k
                drain(k, g)
                gnext = g + (NBUF - 1)

                @pl.when(gnext < NGRP)
                def _():
                    @pl.when(gnext % IDXG == 0)
                    def _():
                        refill(gnext // IDXG)

                    issue((k + NBUF - 1) % NBUF, gnext)

                accum_group(k, g)
            return 0

        lax.fori_loop(0, NGRP // NBUF, outer, 0)
        pltpu.sync_copy(pooled_v, out_hbm.at[pl.ds(wid * SPW, SPW)])

    return pool


def _mlp_call(pooled, W1, b1r, gammar, betar, W2, b2r, B, D, H, CLS, blk):
    nblk = B // blk

    def body(p_ref, w1_ref, b1_ref, g_ref, be_ref, w2_ref, b2_ref,
             o_ref, sum_scr, s_scr, muh_scr, scale_scr):
        ph = pl.program_id(0)
        j = pl.program_id(1)

        @pl.when(ph == 0)
        def _():
            @pl.when(j == 0)
            def _():
                sum_scr[...] = jnp.zeros_like(sum_scr)
                s_scr[...] = jnp.zeros_like(s_scr)

            p = p_ref[...]
            sum_scr[...] += jnp.sum(p, axis=0, keepdims=True)
            s_scr[...] += lax.dot_general(
                p, p, (((0,), (0,)), ((), ())),
                preferred_element_type=jnp.float32)

            @pl.when(j == nblk - 1)
            def _():
                mu_p = sum_scr[...] * (1.0 / B)
                cov = s_scr[...] * (1.0 / B) - lax.dot_general(
                    mu_p, mu_p, (((0,), (0,)), ((), ())),
                    preferred_element_type=jnp.float32)
                w1 = w1_ref[...]
                muh_scr[...] = (
                    jnp.dot(mu_p, w1, preferred_element_type=jnp.float32)
                    + b1_ref[...])
                m = jnp.dot(cov, w1, preferred_element_type=jnp.float32)
                varh = jnp.sum(w1 * m, axis=0, keepdims=True)
                scale_scr[...] = g_ref[...] * lax.rsqrt(varh + EPS)

        @pl.when(ph == 1)
        def _():
            h = jnp.dot(p_ref[...], w1_ref[...],
                        preferred_element_type=jnp.float32) + b1_ref[...]
            hn = (h - muh_scr[...]) * scale_scr[...] + be_ref[...]
            hr = jnp.maximum(hn, 0.0)
            o_ref[...] = jnp.dot(
                hr, w2_ref[...], preferred_element_type=jnp.float32) + b2_ref[...]

    full = lambda ph, j: (0, 0)
    return pl.pallas_call(
        body,
        grid=(2, nblk),
        in_specs=[
            pl.BlockSpec((blk, D), lambda ph, j: (j, 0)),
            pl.BlockSpec((D, H), full),
            pl.BlockSpec((1, H), full),
            pl.BlockSpec((1, H), full),
            pl.BlockSpec((1, H), full),
            pl.BlockSpec((H, CLS), full),
            pl.BlockSpec((1, CLS), full),
        ],
        out_specs=pl.BlockSpec((blk, CLS), lambda ph, j: (j, 0)),
        out_shape=jax.ShapeDtypeStruct((B, CLS), jnp.float32),
        scratch_shapes=[
            pltpu.VMEM((1, D), jnp.float32),
            pltpu.VMEM((D, D), jnp.float32),
            pltpu.VMEM((1, H), jnp.float32),
            pltpu.VMEM((1, H), jnp.float32),
        ],
    )(pooled, W1, b1r, gammar, betar, W2, b2r)


def kernel(x, embed, W1, b1, gamma, beta, W2, b2):
    B, SEQ = x.shape
    V, D = embed.shape
    H = W1.shape[1]
    CLS = W2.shape[1]

    embed_pk = _pack_kernel(V, D)(embed)
    pooled = _pool_kernel(B, SEQ, V, D)(x.astype(jnp.int32), embed_pk)

    out = _mlp_call(
        pooled, W1, b1.reshape(1, H), gamma.reshape(1, H),
        beta.reshape(1, H), W2, b2.reshape(1, CLS), B, D, H, CLS, blk=4096)
    return out
